# baseline (speedup 1.0000x reference)
import functools

import jax
import jax.numpy as jnp
from jax import lax
from jax.experimental import pallas as pl
from jax.experimental.pallas import tpu as pltpu

_BK = 8192
_NSLOTS = 6
_WIN = 1024


def _body(q_ref, db_hbm, cat_hbm, shp_hbm, cat_out_ref, shp_out_ref,
          buf_ref, qn_ref, bv_ref, bi_ref, gbuf_ref, bi_smem, sem, gsem,
          *, bk, k_total, nblk, last_bk, last_valid, nq):
    i = pl.program_id(0)

    def start_copy(j, slot):
        @pl.when(j < nblk - 1)
        def _full():
            pltpu.make_async_copy(
                db_hbm.at[:, pl.ds(j * bk, bk)], buf_ref.at[slot],
                sem.at[slot]).start()

        @pl.when(j == nblk - 1)
        def _last():
            pltpu.make_async_copy(
                db_hbm.at[:, pl.ds(j * bk, last_bk)],
                buf_ref.at[slot, :, pl.ds(0, last_bk)], sem.at[slot]).start()

    @pl.when(i == 0)
    def _init():
        for j in range(min(_NSLOTS - 1, nblk)):
            start_copy(jnp.int32(j), jnp.int32(j))
        bv_ref[...] = jnp.full_like(bv_ref, -jnp.inf)
        bi_ref[...] = jnp.zeros_like(bi_ref)
        q = q_ref[...]
        norm = jnp.sqrt(jnp.sum(q * q, axis=1, keepdims=True))
        qn_ref[...] = q / jnp.clip(norm, 1e-12)

    nxt = i + _NSLOTS - 1

    @pl.when(nxt < nblk)
    def _prefetch():
        start_copy(nxt, lax.rem(nxt, _NSLOTS))

    def _update(sim, base):
        m = jnp.max(sim, axis=1, keepdims=True)
        bv = bv_ref[...]
        better = m > bv

        @pl.when(jnp.any(better))
        def _():
            iota = lax.broadcasted_iota(jnp.int32, sim.shape, 1)
            li = jnp.min(jnp.where(sim == m, iota, k_total), axis=1,
                         keepdims=True)
            bi_ref[...] = jnp.where(better, base + li, bi_ref[...])
            bv_ref[...] = jnp.where(better, m, bv)

    slot = lax.rem(i, _NSLOTS)

    @pl.when(i < nblk - 1)
    def _main():
        pltpu.make_async_copy(
            db_hbm.at[:, pl.ds(i * bk, bk)], buf_ref.at[slot],
            sem.at[slot]).wait()
        sim = lax.dot_general(
            qn_ref[...], buf_ref[slot],
            (((1,), (0,)), ((), ())),
            preferred_element_type=jnp.float32,
        )
        _update(sim, i * bk)

    @pl.when(i == nblk - 1)
    def _lastblk():
        pltpu.make_async_copy(
            db_hbm.at[:, pl.ds(i * bk, last_bk)],
            buf_ref.at[slot, :, pl.ds(0, last_bk)], sem.at[slot]).wait()
        sim = lax.dot_general(
            qn_ref[...], buf_ref[slot, :, pl.ds(0, last_bk)],
            (((1,), (0,)), ((), ())),
            preferred_element_type=jnp.float32,
        )
        if last_valid < last_bk:
            lane = lax.broadcasted_iota(jnp.int32, sim.shape, 1)
            sim = jnp.where(lane < last_valid, sim, -jnp.inf)
        _update(sim, i * bk)

        pltpu.make_async_copy(bi_ref, bi_smem, gsem.at[0]).start()
        pltpu.make_async_copy(bi_ref, bi_smem, gsem.at[0]).wait()
        for t, tbl in enumerate((cat_hbm, shp_hbm)):
            for qi in range(nq):
                idx_s = bi_smem[qi, 0]
                win = pl.multiple_of((idx_s // _WIN) * _WIN, _WIN)
                pltpu.make_async_copy(
                    tbl.at[pl.ds(win, _WIN)], gbuf_ref.at[t, qi],
                    gsem.at[t]).start()
        for t in range(2):
            for qi in range(nq):
                idx_s = bi_smem[qi, 0]
                win = pl.multiple_of((idx_s // _WIN) * _WIN, _WIN)
                pltpu.make_async_copy(
                    (cat_hbm, shp_hbm)[t].at[pl.ds(win, _WIN)],
                    gbuf_ref.at[t, qi], gsem.at[t]).wait()
        off = lax.rem(bi_ref[...], _WIN)
        lane = lax.broadcasted_iota(jnp.int32, (nq, _WIN), 1)
        hit = lane == off
        for t, out_ref in enumerate((cat_out_ref, shp_out_ref)):
            vals = jnp.sum(jnp.where(hit, gbuf_ref[t], 0), axis=1)
            out_ref[...] = vals


def _retrieve(q, dbt, cat, shp, bk, interpret=False):
    d, k_total = dbt.shape
    nq = q.shape[0]
    nblk = (k_total + bk - 1) // bk
    last_valid = k_total - (nblk - 1) * bk
    last_bk = ((last_valid + 127) // 128) * 128
    return pl.pallas_call(
        functools.partial(_body, bk=bk, k_total=k_total, nblk=nblk,
                          last_bk=last_bk, last_valid=last_valid, nq=nq),
        grid=(nblk,),
        in_specs=[
            pl.BlockSpec((nq, d), lambda i: (0, 0)),
            pl.BlockSpec(memory_space=pltpu.MemorySpace.HBM),
            pl.BlockSpec(memory_space=pltpu.MemorySpace.HBM),
            pl.BlockSpec(memory_space=pltpu.MemorySpace.HBM),
        ],
        out_specs=[
            pl.BlockSpec((nq,), lambda i: (0,)),
            pl.BlockSpec((nq,), lambda i: (0,)),
        ],
        out_shape=[
            jax.ShapeDtypeStruct((nq,), jnp.int32),
            jax.ShapeDtypeStruct((nq,), jnp.int32),
        ],
        scratch_shapes=[
            pltpu.VMEM((_NSLOTS, d, bk), jnp.float32),
            pltpu.VMEM((nq, d), jnp.float32),
            pltpu.VMEM((nq, 1), jnp.float32),
            pltpu.VMEM((nq, 1), jnp.int32),
            pltpu.VMEM((2, nq, _WIN), jnp.int32),
            pltpu.SMEM((nq, 1), jnp.int32),
            pltpu.SemaphoreType.DMA((_NSLOTS,)),
            pltpu.SemaphoreType.DMA((2,)),
        ],
        compiler_params=pltpu.CompilerParams(
            dimension_semantics=("arbitrary",),
        ),
        interpret=interpret,
    )(q, dbt, cat, shp)


def kernel(shape_embedding, db_embedding, category_idx, shape_idx):
    cat, shp = _retrieve(shape_embedding, db_embedding.T, category_idx,
                         shape_idx, _BK)
    return cat, shp


# speedup vs baseline: 1.1078x; 1.1078x over previous
import functools

import jax
import jax.numpy as jnp
from jax import lax
from jax.experimental import pallas as pl
from jax.experimental.pallas import tpu as pltpu

_BK = 16384
_NSLOTS = 6
_WIN = 1024


def _body(q_ref, db_hbm, cat_hbm, shp_hbm, cat_out_ref, shp_out_ref,
          buf_ref, qn_ref, bv_ref, bi_ref, gbuf_ref, bi_smem, sem, gsem,
          *, bk, k_total, nblk, last_bk, last_valid, nq):
    i = pl.program_id(0)

    def start_copy(j, slot):
        @pl.when(j < nblk - 1)
        def _full():
            pltpu.make_async_copy(
                db_hbm.at[:, pl.ds(j * bk, bk)], buf_ref.at[slot],
                sem.at[slot]).start()

        @pl.when(j == nblk - 1)
        def _last():
            pltpu.make_async_copy(
                db_hbm.at[:, pl.ds(j * bk, last_bk)],
                buf_ref.at[slot, :, pl.ds(0, last_bk)], sem.at[slot]).start()

    @pl.when(i == 0)
    def _init():
        for j in range(min(_NSLOTS - 1, nblk)):
            start_copy(jnp.int32(j), jnp.int32(j))
        bv_ref[...] = jnp.full_like(bv_ref, -jnp.inf)
        bi_ref[...] = jnp.zeros_like(bi_ref)
        q = q_ref[...]
        norm = jnp.sqrt(jnp.sum(q * q, axis=1, keepdims=True))
        qn_ref[...] = q / jnp.clip(norm, 1e-12)

    nxt = i + _NSLOTS - 1

    @pl.when(nxt < nblk)
    def _prefetch():
        start_copy(nxt, lax.rem(nxt, _NSLOTS))

    def _update(sim, base):
        m = jnp.max(sim, axis=1, keepdims=True)
        bv = bv_ref[...]
        better = m > bv

        @pl.when(jnp.any(better))
        def _():
            iota = lax.broadcasted_iota(jnp.int32, sim.shape, 1)
            li = jnp.min(jnp.where(sim == m, iota, k_total), axis=1,
                         keepdims=True)
            bi_ref[...] = jnp.where(better, base + li, bi_ref[...])
            bv_ref[...] = jnp.where(better, m, bv)

    slot = lax.rem(i, _NSLOTS)

    @pl.when(i < nblk - 1)
    def _main():
        pltpu.make_async_copy(
            db_hbm.at[:, pl.ds(i * bk, bk)], buf_ref.at[slot],
            sem.at[slot]).wait()
        sim = lax.dot_general(
            qn_ref[...], buf_ref[slot],
            (((1,), (0,)), ((), ())),
            preferred_element_type=jnp.float32,
        )
        _update(sim, i * bk)

    @pl.when(i == nblk - 1)
    def _lastblk():
        pltpu.make_async_copy(
            db_hbm.at[:, pl.ds(i * bk, last_bk)],
            buf_ref.at[slot, :, pl.ds(0, last_bk)], sem.at[slot]).wait()
        sim = lax.dot_general(
            qn_ref[...], buf_ref[slot, :, pl.ds(0, last_bk)],
            (((1,), (0,)), ((), ())),
            preferred_element_type=jnp.float32,
        )
        if last_valid < last_bk:
            lane = lax.broadcasted_iota(jnp.int32, sim.shape, 1)
            sim = jnp.where(lane < last_valid, sim, -jnp.inf)
        _update(sim, i * bk)

        pltpu.make_async_copy(bi_ref, bi_smem, gsem.at[0]).start()
        pltpu.make_async_copy(bi_ref, bi_smem, gsem.at[0]).wait()
        for t, tbl in enumerate((cat_hbm, shp_hbm)):
            for qi in range(nq):
                idx_s = bi_smem[qi, 0]
                win = pl.multiple_of((idx_s // _WIN) * _WIN, _WIN)
                pltpu.make_async_copy(
                    tbl.at[pl.ds(win, _WIN)], gbuf_ref.at[t, qi],
                    gsem.at[t]).start()
        for t in range(2):
            for qi in range(nq):
                idx_s = bi_smem[qi, 0]
                win = pl.multiple_of((idx_s // _WIN) * _WIN, _WIN)
                pltpu.make_async_copy(
                    (cat_hbm, shp_hbm)[t].at[pl.ds(win, _WIN)],
                    gbuf_ref.at[t, qi], gsem.at[t]).wait()
        off = lax.rem(bi_ref[...], _WIN)
        lane = lax.broadcasted_iota(jnp.int32, (nq, _WIN), 1)
        hit = lane == off
        for t, out_ref in enumerate((cat_out_ref, shp_out_ref)):
            vals = jnp.sum(jnp.where(hit, gbuf_ref[t], 0), axis=1)
            out_ref[...] = vals


def _retrieve(q, dbt, cat, shp, bk, interpret=False):
    d, k_total = dbt.shape
    nq = q.shape[0]
    nblk = (k_total + bk - 1) // bk
    last_valid = k_total - (nblk - 1) * bk
    last_bk = ((last_valid + 127) // 128) * 128
    return pl.pallas_call(
        functools.partial(_body, bk=bk, k_total=k_total, nblk=nblk,
                          last_bk=last_bk, last_valid=last_valid, nq=nq),
        grid=(nblk,),
        in_specs=[
            pl.BlockSpec((nq, d), lambda i: (0, 0)),
            pl.BlockSpec(memory_space=pltpu.MemorySpace.HBM),
            pl.BlockSpec(memory_space=pltpu.MemorySpace.HBM),
            pl.BlockSpec(memory_space=pltpu.MemorySpace.HBM),
        ],
        out_specs=[
            pl.BlockSpec((nq,), lambda i: (0,)),
            pl.BlockSpec((nq,), lambda i: (0,)),
        ],
        out_shape=[
            jax.ShapeDtypeStruct((nq,), jnp.int32),
            jax.ShapeDtypeStruct((nq,), jnp.int32),
        ],
        scratch_shapes=[
            pltpu.VMEM((_NSLOTS, d, bk), jnp.float32),
            pltpu.VMEM((nq, d), jnp.float32),
            pltpu.VMEM((nq, 1), jnp.float32),
            pltpu.VMEM((nq, 1), jnp.int32),
            pltpu.VMEM((2, nq, _WIN), jnp.int32),
            pltpu.SMEM((nq, 1), jnp.int32),
            pltpu.SemaphoreType.DMA((_NSLOTS,)),
            pltpu.SemaphoreType.DMA((2,)),
        ],
        compiler_params=pltpu.CompilerParams(
            dimension_semantics=("arbitrary",),
        ),
        interpret=interpret,
    )(q, dbt, cat, shp)


def kernel(shape_embedding, db_embedding, category_idx, shape_idx):
    cat, shp = _retrieve(shape_embedding, db_embedding.T, category_idx,
                         shape_idx, _BK)
    return cat, shp


# speedup vs baseline: 1.1340x; 1.0237x over previous
import functools

import jax
import jax.numpy as jnp
from jax import lax
from jax.experimental import pallas as pl
from jax.experimental.pallas import tpu as pltpu

_BK = 16384
_NSLOTS = 3
_WIN = 1024


def _body(q_ref, db_hbm, cat_hbm, shp_hbm, cat_out_ref, shp_out_ref,
          buf_ref, qn_ref, bv_ref, bi_ref, gbuf_ref, bi_smem, sem, gsem,
          *, bk, k_total, nblk, last_bk, last_valid, nq):
    i = pl.program_id(0)

    def start_copy(j, slot):
        @pl.when(j < nblk - 1)
        def _full():
            pltpu.make_async_copy(
                db_hbm.at[:, pl.ds(j * bk, bk)], buf_ref.at[slot],
                sem.at[slot]).start()

        @pl.when(j == nblk - 1)
        def _last():
            pltpu.make_async_copy(
                db_hbm.at[:, pl.ds(j * bk, last_bk)],
                buf_ref.at[slot, :, pl.ds(0, last_bk)], sem.at[slot]).start()

    @pl.when(i == 0)
    def _init():
        for j in range(min(_NSLOTS - 1, nblk)):
            start_copy(jnp.int32(j), jnp.int32(j))
        bv_ref[...] = jnp.full_like(bv_ref, -jnp.inf)
        bi_ref[...] = jnp.zeros_like(bi_ref)
        q = q_ref[...]
        norm = jnp.sqrt(jnp.sum(q * q, axis=1, keepdims=True))
        qn_ref[...] = q / jnp.clip(norm, 1e-12)

    nxt = i + _NSLOTS - 1

    @pl.when(nxt < nblk)
    def _prefetch():
        start_copy(nxt, lax.rem(nxt, _NSLOTS))

    def _update(sim, base):
        m = jnp.max(sim, axis=1, keepdims=True)
        bv = bv_ref[...]
        better = m > bv

        @pl.when(jnp.any(better))
        def _():
            iota = lax.broadcasted_iota(jnp.int32, sim.shape, 1)
            li = jnp.min(jnp.where(sim == m, iota, k_total), axis=1,
                         keepdims=True)
            bi_ref[...] = jnp.where(better, base + li, bi_ref[...])
            bv_ref[...] = jnp.where(better, m, bv)

    slot = lax.rem(i, _NSLOTS)

    @pl.when(i < nblk - 1)
    def _main():
        pltpu.make_async_copy(
            db_hbm.at[:, pl.ds(i * bk, bk)], buf_ref.at[slot],
            sem.at[slot]).wait()
        sim = lax.dot_general(
            qn_ref[...], buf_ref[slot],
            (((1,), (0,)), ((), ())),
            preferred_element_type=jnp.float32,
        )
        _update(sim, i * bk)

    @pl.when(i == nblk - 1)
    def _lastblk():
        pltpu.make_async_copy(
            db_hbm.at[:, pl.ds(i * bk, last_bk)],
            buf_ref.at[slot, :, pl.ds(0, last_bk)], sem.at[slot]).wait()
        sim = lax.dot_general(
            qn_ref[...], buf_ref[slot, :, pl.ds(0, last_bk)],
            (((1,), (0,)), ((), ())),
            preferred_element_type=jnp.float32,
        )
        if last_valid < last_bk:
            lane = lax.broadcasted_iota(jnp.int32, sim.shape, 1)
            sim = jnp.where(lane < last_valid, sim, -jnp.inf)
        _update(sim, i * bk)

        pltpu.make_async_copy(bi_ref, bi_smem, gsem.at[0]).start()
        pltpu.make_async_copy(bi_ref, bi_smem, gsem.at[0]).wait()
        for t, tbl in enumerate((cat_hbm, shp_hbm)):
            for qi in range(nq):
                idx_s = bi_smem[qi, 0]
                win = pl.multiple_of((idx_s // _WIN) * _WIN, _WIN)
                pltpu.make_async_copy(
                    tbl.at[pl.ds(win, _WIN)], gbuf_ref.at[t, qi],
                    gsem.at[t]).start()
        for t in range(2):
            for qi in range(nq):
                idx_s = bi_smem[qi, 0]
                win = pl.multiple_of((idx_s // _WIN) * _WIN, _WIN)
                pltpu.make_async_copy(
                    (cat_hbm, shp_hbm)[t].at[pl.ds(win, _WIN)],
                    gbuf_ref.at[t, qi], gsem.at[t]).wait()
        off = lax.rem(bi_ref[...], _WIN)
        lane = lax.broadcasted_iota(jnp.int32, (nq, _WIN), 1)
        hit = lane == off
        for t, out_ref in enumerate((cat_out_ref, shp_out_ref)):
            vals = jnp.sum(jnp.where(hit, gbuf_ref[t], 0), axis=1)
            out_ref[...] = vals


def _retrieve(q, dbt, cat, shp, bk, interpret=False):
    d, k_total = dbt.shape
    nq = q.shape[0]
    nblk = (k_total + bk - 1) // bk
    last_valid = k_total - (nblk - 1) * bk
    last_bk = ((last_valid + 127) // 128) * 128
    return pl.pallas_call(
        functools.partial(_body, bk=bk, k_total=k_total, nblk=nblk,
                          last_bk=last_bk, last_valid=last_valid, nq=nq),
        grid=(nblk,),
        in_specs=[
            pl.BlockSpec((nq, d), lambda i: (0, 0)),
            pl.BlockSpec(memory_space=pltpu.MemorySpace.HBM),
            pl.BlockSpec(memory_space=pltpu.MemorySpace.HBM),
            pl.BlockSpec(memory_space=pltpu.MemorySpace.HBM),
        ],
        out_specs=[
            pl.BlockSpec((nq,), lambda i: (0,)),
            pl.BlockSpec((nq,), lambda i: (0,)),
        ],
        out_shape=[
            jax.ShapeDtypeStruct((nq,), jnp.int32),
            jax.ShapeDtypeStruct((nq,), jnp.int32),
        ],
        scratch_shapes=[
            pltpu.VMEM((_NSLOTS, d, bk), jnp.float32),
            pltpu.VMEM((nq, d), jnp.float32),
            pltpu.VMEM((nq, 1), jnp.float32),
            pltpu.VMEM((nq, 1), jnp.int32),
            pltpu.VMEM((2, nq, _WIN), jnp.int32),
            pltpu.SMEM((nq, 1), jnp.int32),
            pltpu.SemaphoreType.DMA((_NSLOTS,)),
            pltpu.SemaphoreType.DMA((2,)),
        ],
        compiler_params=pltpu.CompilerParams(
            dimension_semantics=("arbitrary",),
        ),
        interpret=interpret,
    )(q, dbt, cat, shp)


def kernel(shape_embedding, db_embedding, category_idx, shape_idx):
    cat, shp = _retrieve(shape_embedding, db_embedding.T, category_idx,
                         shape_idx, _BK)
    return cat, shp
